# baseline (speedup 1.0000x reference)
import jax
import jax.numpy as jnp
from jax.experimental import pallas as pl
from jax.experimental.pallas import tpu as pltpu

B, T, N, H = 4, 16, 10000, 64
HS = 2
HB = H // HS


def _body(sp_ref, stack_ref, hid_ref, hint_ref, spv_ref, out_ref, ptr_ref):
    b = pl.program_id(0)
    t = pl.program_id(2)
    tgt = sp_ref[b] + 1

    @pl.when(t == tgt)
    def _():
        out_ref[...] = hid_ref[...]

    @pl.when(t != tgt)
    def _():
        out_ref[...] = stack_ref[...]

    @pl.when((b == 0) & (pl.program_id(1) == 0) & (t == 0))
    def _():
        h = hint_ref[...]
        a0 = h[:, :, 0]
        a1 = h[:, :, 1]
        a2 = h[:, :, 2]
        ops = jnp.where(a0 >= a1,
                        jnp.where(a0 >= a2, 0, 2),
                        jnp.where(a1 >= a2, 1, 2)).astype(jnp.int32)
        ptr_ref[...] = jnp.maximum(spv_ref[...] + ops - 1, 0)


def kernel(stack, stack_pointers, hint_preds, hiddens, graph_fts):
    del graph_fts
    sp_flat = jnp.reshape(stack_pointers, (B,))
    stack_t = jnp.transpose(stack, (0, 1, 3, 2))
    hid_t = jnp.transpose(hiddens, (0, 1, 3, 2))

    def stack_idx(b, h, t, sp):
        tt = jnp.where(t == sp[b] + 1, t - 1, t)
        return (b, tt, h, 0)

    grid_spec = pltpu.PrefetchScalarGridSpec(
        num_scalar_prefetch=1,
        grid=(B, HS, T),
        in_specs=[
            pl.BlockSpec((1, 1, HB, N), stack_idx),
            pl.BlockSpec((1, 1, HB, N), lambda b, h, t, sp: (0, b, h, 0)),
            pl.BlockSpec((1, B, 3), lambda b, h, t, sp: (0, 0, 0)),
            pl.BlockSpec((1, B), lambda b, h, t, sp: (0, 0)),
        ],
        out_specs=[
            pl.BlockSpec((1, 1, HB, N), lambda b, h, t, sp: (b, t, h, 0)),
            pl.BlockSpec((1, B), lambda b, h, t, sp: (0, 0)),
        ],
    )

    new_stack_t, new_ptrs = pl.pallas_call(
        _body,
        grid_spec=grid_spec,
        out_shape=[
            jax.ShapeDtypeStruct((B, T, H, N), jnp.float32),
            jax.ShapeDtypeStruct((1, B), jnp.int32),
        ],
    )(sp_flat, stack_t, hid_t, hint_preds, stack_pointers)
    return (jnp.transpose(new_stack_t, (0, 1, 3, 2)), new_ptrs)


# speedup vs baseline: 1.3510x; 1.3510x over previous
import jax
import jax.numpy as jnp
from jax.experimental import pallas as pl
from jax.experimental.pallas import tpu as pltpu

B, T, N, H = 4, 16, 10000, 64
TB = 2
TT = T // TB


def _body(sp_ref, stack_ref, hid_ref, hint_ref, spv_ref, out_ref, ptr_ref):
    b = pl.program_id(0)
    tb = pl.program_id(1)
    tgt = sp_ref[b] + 1
    base = TB * tb

    out_ref[...] = stack_ref[...]

    @pl.when((tgt >= base) & (tgt < base + TB))
    def _():
        out_ref[:, pl.ds(tgt - base, 1)] = hid_ref[...]

    @pl.when((b == 0) & (tb == 0))
    def _():
        h = hint_ref[...]
        a0 = h[:, :, 0]
        a1 = h[:, :, 1]
        a2 = h[:, :, 2]
        ops = jnp.where(a0 >= a1,
                        jnp.where(a0 >= a2, 0, 2),
                        jnp.where(a1 >= a2, 1, 2)).astype(jnp.int32)
        ptr_ref[...] = jnp.maximum(spv_ref[...] + ops - 1, 0)


def kernel(stack, stack_pointers, hint_preds, hiddens, graph_fts):
    del graph_fts
    sp_flat = jnp.reshape(stack_pointers, (B,))
    stack_t = jnp.transpose(stack, (0, 1, 3, 2))
    hid_t = jnp.transpose(hiddens, (0, 1, 3, 2))

    grid_spec = pltpu.PrefetchScalarGridSpec(
        num_scalar_prefetch=1,
        grid=(B, TT),
        in_specs=[
            pl.BlockSpec((1, TB, H, N), lambda b, tb, sp: (b, tb, 0, 0)),
            pl.BlockSpec((1, 1, H, N), lambda b, tb, sp: (0, b, 0, 0)),
            pl.BlockSpec((1, B, 3), lambda b, tb, sp: (0, 0, 0)),
            pl.BlockSpec((1, B), lambda b, tb, sp: (0, 0)),
        ],
        out_specs=[
            pl.BlockSpec((1, TB, H, N), lambda b, tb, sp: (b, tb, 0, 0)),
            pl.BlockSpec((1, B), lambda b, tb, sp: (0, 0)),
        ],
    )

    new_stack_t, new_ptrs = pl.pallas_call(
        _body,
        grid_spec=grid_spec,
        out_shape=[
            jax.ShapeDtypeStruct((B, T, H, N), jnp.float32),
            jax.ShapeDtypeStruct((1, B), jnp.int32),
        ],
    )(sp_flat, stack_t, hid_t, hint_preds, stack_pointers)
    return (jnp.transpose(new_stack_t, (0, 1, 3, 2)), new_ptrs)


# speedup vs baseline: 1.3748x; 1.0176x over previous
import jax
import jax.numpy as jnp
from jax.experimental import pallas as pl
from jax.experimental.pallas import tpu as pltpu

B, T, N, H = 4, 16, 10000, 64
TB = 4
TT = T // TB


def _body(sp_ref, stack_ref, hid_ref, hint_ref, spv_ref, out_ref, ptr_ref):
    b = pl.program_id(0)
    tb = pl.program_id(1)
    tgt = sp_ref[b] + 1
    base = TB * tb

    out_ref[...] = stack_ref[...]

    @pl.when((tgt >= base) & (tgt < base + TB))
    def _():
        out_ref[:, pl.ds(tgt - base, 1)] = hid_ref[...]

    @pl.when((b == 0) & (tb == 0))
    def _():
        h = hint_ref[...]
        a0 = h[:, :, 0]
        a1 = h[:, :, 1]
        a2 = h[:, :, 2]
        ops = jnp.where(a0 >= a1,
                        jnp.where(a0 >= a2, 0, 2),
                        jnp.where(a1 >= a2, 1, 2)).astype(jnp.int32)
        ptr_ref[...] = jnp.maximum(spv_ref[...] + ops - 1, 0)


def kernel(stack, stack_pointers, hint_preds, hiddens, graph_fts):
    del graph_fts
    sp_flat = jnp.reshape(stack_pointers, (B,))
    stack_t = jnp.transpose(stack, (0, 1, 3, 2))
    hid_t = jnp.transpose(hiddens, (0, 1, 3, 2))

    grid_spec = pltpu.PrefetchScalarGridSpec(
        num_scalar_prefetch=1,
        grid=(B, TT),
        in_specs=[
            pl.BlockSpec((1, TB, H, N), lambda b, tb, sp: (b, tb, 0, 0)),
            pl.BlockSpec((1, 1, H, N), lambda b, tb, sp: (0, b, 0, 0)),
            pl.BlockSpec((1, B, 3), lambda b, tb, sp: (0, 0, 0)),
            pl.BlockSpec((1, B), lambda b, tb, sp: (0, 0)),
        ],
        out_specs=[
            pl.BlockSpec((1, TB, H, N), lambda b, tb, sp: (b, tb, 0, 0)),
            pl.BlockSpec((1, B), lambda b, tb, sp: (0, 0)),
        ],
    )

    new_stack_t, new_ptrs = pl.pallas_call(
        _body,
        grid_spec=grid_spec,
        out_shape=[
            jax.ShapeDtypeStruct((B, T, H, N), jnp.float32),
            jax.ShapeDtypeStruct((1, B), jnp.int32),
        ],
    )(sp_flat, stack_t, hid_t, hint_preds, stack_pointers)
    return (jnp.transpose(new_stack_t, (0, 1, 3, 2)), new_ptrs)
